# baseline (speedup 1.0000x reference)
import dataclasses
import functools

import jax
import jax.numpy as jnp
from jax import lax
from jax.experimental import pallas as pl
from jax.experimental.pallas import tpu as pltpu
from jax.experimental.pallas import tpu_sc as plsc

N_PAD = 10240
R = 1024
NF = 128
TROWS = 136
BLK = 800
GRP = 5
FPW = 4


def _leaky_bound(m):
    return jnp.maximum(m, 0.2 * m)


def _pack_rows(xsT):
    lo = lax.bitcast_convert_type(
        xsT[: NF // 2, :].astype(jnp.bfloat16), jnp.uint16).astype(jnp.uint32)
    hi = lax.bitcast_convert_type(
        xsT[NF // 2:, :].astype(jnp.bfloat16), jnp.uint16).astype(jnp.uint32)
    return lax.bitcast_convert_type((hi << 16) | lo, jnp.int32)


def _pre_body(xT, Wsrc, Wdst, asrc, adst, Wl, bl, tab, att, skt, bmx):
    i = pl.program_id(0)
    xTb = xT[...]
    xsT = lax.dot_general(Wsrc[...], xTb, (((0,), (0,)), ((), ())),
                          preferred_element_type=jnp.float32)
    a_s = jnp.dot(asrc[...], xsT, preferred_element_type=jnp.float32)
    u = lax.dot_general(adst[...], Wdst[...], (((1,), (1,)), ((), ())),
                        preferred_element_type=jnp.float32)
    a_d = jnp.dot(u, xTb, preferred_element_type=jnp.float32)
    tab[...] = _pack_rows(xsT)
    att[...] = jnp.concatenate(
        [a_s, a_d, jnp.zeros((6, R), jnp.float32)], axis=0)
    skt[...] = lax.dot_general(Wl[...], xTb, (((0,), (0,)), ((), ())),
                               preferred_element_type=jnp.float32) + bl[...]

    @pl.when(i == 0)
    def _():
        bmx[...] = jnp.full((8, 128), -jnp.inf, jnp.float32)

    ms = jnp.max(a_s)
    md = jnp.max(a_d)
    bmx[0:1, :] = jnp.maximum(bmx[0:1, :], ms)
    bmx[1:2, :] = jnp.maximum(bmx[1:2, :], md)

    @pl.when(i == pl.num_programs(0) - 1)
    def _():
        m = bmx[0:1, :] + bmx[1:2, :]
        bmx[2:3, :] = _leaky_bound(m)


def _tc_pre(xT, Wsrc, Wdst, asrc2d, adst2d, Wl, blcol):
    grid = (N_PAD // R,)
    return pl.pallas_call(
        _pre_body,
        grid=grid,
        in_specs=[
            pl.BlockSpec((NF, R), lambda i: (0, i)),
            pl.BlockSpec((NF, NF), lambda i: (0, 0)),
            pl.BlockSpec((NF, NF), lambda i: (0, 0)),
            pl.BlockSpec((1, NF), lambda i: (0, 0)),
            pl.BlockSpec((1, NF), lambda i: (0, 0)),
            pl.BlockSpec((NF, NF), lambda i: (0, 0)),
            pl.BlockSpec((NF, 1), lambda i: (0, 0)),
        ],
        out_specs=[
            pl.BlockSpec((NF // 2, R), lambda i: (0, i)),
            pl.BlockSpec((8, R), lambda i: (0, i)),
            pl.BlockSpec((NF, R), lambda i: (0, i)),
            pl.BlockSpec((8, 128), lambda i: (0, 0)),
        ],
        out_shape=[
            jax.ShapeDtypeStruct((NF // 2, N_PAD), jnp.int32),
            jax.ShapeDtypeStruct((8, N_PAD), jnp.float32),
            jax.ShapeDtypeStruct((NF, N_PAD), jnp.float32),
            jax.ShapeDtypeStruct((8, 128), jnp.float32),
        ],
    )(xT, Wsrc, Wdst, asrc2d, adst2d, Wl, blcol)


def _mid_body(acc, den, b1, skt1, Wsrc, Wdst, asrc, adst, Wl, bl,
              tab, att, skt, bmx):
    i = pl.program_id(0)
    dinv = 1.0 / (jnp.sum(den[...], axis=0, keepdims=True) + 1e-16)
    hT = jnp.maximum((acc[0] + acc[1]) * dinv + b1[...] + skt1[...], 0.0)
    xsT = lax.dot_general(Wsrc[...], hT, (((0,), (0,)), ((), ())),
                          preferred_element_type=jnp.float32)
    a_s = jnp.dot(asrc[...], xsT, preferred_element_type=jnp.float32)
    u = lax.dot_general(adst[...], Wdst[...], (((1,), (1,)), ((), ())),
                        preferred_element_type=jnp.float32)
    a_d = jnp.dot(u, hT, preferred_element_type=jnp.float32)
    tab[...] = _pack_rows(xsT)
    att[...] = jnp.concatenate(
        [a_s, a_d, jnp.zeros((6, R), jnp.float32)], axis=0)
    skt[...] = lax.dot_general(Wl[...], hT, (((0,), (0,)), ((), ())),
                               preferred_element_type=jnp.float32) + bl[...]

    @pl.when(i == 0)
    def _():
        bmx[...] = jnp.full((8, 128), -jnp.inf, jnp.float32)

    ms = jnp.max(a_s)
    md = jnp.max(a_d)
    bmx[0:1, :] = jnp.maximum(bmx[0:1, :], ms)
    bmx[1:2, :] = jnp.maximum(bmx[1:2, :], md)

    @pl.when(i == pl.num_programs(0) - 1)
    def _():
        m = bmx[0:1, :] + bmx[1:2, :]
        bmx[2:3, :] = _leaky_bound(m)


def _tc_mid(acc1, den1, b1col, skt1, Wsrc, Wdst, asrc2d, adst2d, Wl, blcol):
    grid = (N_PAD // R,)
    return pl.pallas_call(
        _mid_body,
        grid=grid,
        in_specs=[
            pl.BlockSpec((2, NF, R), lambda i: (0, 0, i)),
            pl.BlockSpec((32, R), lambda i: (0, i)),
            pl.BlockSpec((NF, 1), lambda i: (0, 0)),
            pl.BlockSpec((NF, R), lambda i: (0, i)),
            pl.BlockSpec((NF, NF), lambda i: (0, 0)),
            pl.BlockSpec((NF, NF), lambda i: (0, 0)),
            pl.BlockSpec((1, NF), lambda i: (0, 0)),
            pl.BlockSpec((1, NF), lambda i: (0, 0)),
            pl.BlockSpec((NF, NF), lambda i: (0, 0)),
            pl.BlockSpec((NF, 1), lambda i: (0, 0)),
        ],
        out_specs=[
            pl.BlockSpec((NF // 2, R), lambda i: (0, i)),
            pl.BlockSpec((8, R), lambda i: (0, i)),
            pl.BlockSpec((NF, R), lambda i: (0, i)),
            pl.BlockSpec((8, 128), lambda i: (0, 0)),
        ],
        out_shape=[
            jax.ShapeDtypeStruct((NF // 2, N_PAD), jnp.int32),
            jax.ShapeDtypeStruct((8, N_PAD), jnp.float32),
            jax.ShapeDtypeStruct((NF, N_PAD), jnp.float32),
            jax.ShapeDtypeStruct((8, 128), jnp.float32),
        ],
    )(acc1, den1, b1col, skt1, Wsrc, Wdst, asrc2d, adst2d, Wl, blcol)


def _post_body(acc, den, b2, skt2, out):
    dinv = 1.0 / (jnp.sum(den[...], axis=0, keepdims=True) + 1e-16)
    outT = (acc[0] + acc[1]) * dinv + b2[...] + skt2[...]
    out[...] = outT.T


def _tc_post(acc2, den2, b2col, skt2):
    grid = (N_PAD // R,)
    return pl.pallas_call(
        _post_body,
        grid=grid,
        in_specs=[
            pl.BlockSpec((2, NF, R), lambda i: (0, 0, i)),
            pl.BlockSpec((32, R), lambda i: (0, i)),
            pl.BlockSpec((NF, 1), lambda i: (0, 0)),
            pl.BlockSpec((NF, R), lambda i: (0, i)),
        ],
        out_specs=pl.BlockSpec((R, NF), lambda i: (i, 0)),
        out_shape=jax.ShapeDtypeStruct((N_PAD, NF), jnp.float32),
    )(acc2, den2, b2col, skt2)


BLK_A = 2000
GRP_A = 5


def _edge_scalar_pass(att, src, dst, b16):
    E = src.shape[0]
    shard = E // 32
    mesh = plsc.VectorSubcoreMesh(core_axis_name="c", subcore_axis_name="s")
    cp = pltpu.CompilerParams()
    if "needs_layout_passes" in pltpu.CompilerParams.__dataclass_fields__:
        cp = dataclasses.replace(cp, needs_layout_passes=False)

    @functools.partial(
        pl.kernel,
        out_type=(jax.ShapeDtypeStruct((E,), jnp.float32),
                  jax.ShapeDtypeStruct((E,), jnp.int32),
                  jax.ShapeDtypeStruct((32, N_PAD), jnp.float32)),
        mesh=mesh,
        compiler_params=cp,
        scratch_types=[
            pltpu.VMEM((N_PAD,), jnp.float32),
            pltpu.VMEM((N_PAD,), jnp.float32),
            pltpu.VMEM((N_PAD,), jnp.float32),
            pltpu.VMEM((shard,), jnp.float32),
            pltpu.VMEM((shard,), jnp.int32),
            pltpu.VMEM((16,), jnp.float32),
            pltpu.VMEM((BLK_A,), jnp.int32),
            pltpu.VMEM((BLK_A,), jnp.int32),
            pltpu.VMEM((BLK_A,), jnp.int32),
            pltpu.VMEM((BLK_A,), jnp.int32),
            pltpu.SemaphoreType.DMA,
            pltpu.SemaphoreType.DMA,
            pltpu.SemaphoreType.DMA,
            pltpu.SemaphoreType.DMA,
        ],
    )
    def k(att_hbm, src_hbm, dst_hbm, b_hbm, ex_hbm, ids_hbm, denp_hbm,
          asrc_v, adst_v, den_v, ex_v, ids_v, b_v, s0, s1, d0, d1,
          sem0, sem1, sem2, sem3):
        wid = lax.axis_index("s") * 2 + lax.axis_index("c")
        ebase = wid * shard
        pltpu.sync_copy(att_hbm.at[0], asrc_v)
        pltpu.sync_copy(att_hbm.at[1], adst_v)
        pltpu.sync_copy(b_hbm, b_v)

        zz = jnp.zeros((16,), jnp.float32)

        @pl.loop(0, N_PAD // 16, unroll=4)
        def _(i):
            den_v[pl.ds(i * 16, 16)] = zz

        bvec = b_v[...]
        nblk = shard // BLK_A

        def start(bs, bd, blk, sa, sb):
            off = ebase + blk * BLK_A
            pltpu.async_copy(src_hbm.at[pl.ds(off, BLK_A)], bs, sa)
            pltpu.async_copy(dst_hbm.at[pl.ds(off, BLK_A)], bd, sb)

        def wait(bs, bd, sa, sb):
            pltpu.make_async_copy(src_hbm.at[pl.ds(0, BLK_A)], bs, sa).wait()
            pltpu.make_async_copy(dst_hbm.at[pl.ds(0, BLK_A)], bd, sb).wait()

        def process(bs, bd, blk):
            xbase = blk * BLK_A

            @plsc.parallel_loop(0, BLK_A // 16 // GRP_A)
            def _(cc):
                c0 = cc * GRP_A
                sis = [bs[pl.ds((c0 + g) * 16, 16)] for g in range(GRP_A)]
                dis = [bd[pl.ds((c0 + g) * 16, 16)] for g in range(GRP_A)]
                aas = [plsc.load_gather(asrc_v, [sis[g]]) for g in range(GRP_A)]
                ads = [plsc.load_gather(adst_v, [dis[g]]) for g in range(GRP_A)]
                als = [aas[g] + ads[g] for g in range(GRP_A)]
                als = [jnp.maximum(al, al * 0.2) - bvec for al in als]
                es = [jnp.exp(al) for al in als]
                for g in range(GRP_A):
                    plsc.addupdate_scatter(den_v, [dis[g]], es[g])
                    ex_v[pl.ds(xbase + (c0 + g) * 16, 16)] = es[g]
                    ids_v[pl.ds(xbase + (c0 + g) * 16, 16)] = (
                        sis[g] | (dis[g] << 16))

        start(s0, d0, 0, sem0, sem1)
        start(s1, d1, 1, sem2, sem3)

        @pl.loop(0, nblk // 2)
        def _(p):
            g = p * 2
            wait(s0, d0, sem0, sem1)
            process(s0, d0, g)
            start(s0, d0, lax.rem(g + 2, nblk), sem0, sem1)
            wait(s1, d1, sem2, sem3)
            process(s1, d1, g + 1)
            start(s1, d1, lax.rem(g + 3, nblk), sem2, sem3)

        wait(s0, d0, sem0, sem1)
        if nblk % 2 == 1:
            process(s0, d0, nblk - 1)
        wait(s1, d1, sem2, sem3)

        pltpu.sync_copy(ex_v, ex_hbm.at[pl.ds(ebase, shard)])
        pltpu.sync_copy(ids_v, ids_hbm.at[pl.ds(ebase, shard)])
        pltpu.sync_copy(den_v, denp_hbm.at[wid])

    return k(att, src, dst, b16)


def _edge_pass(tabp, ids, ex):
    E = ids.shape[0]
    half = E // 2
    nblk = half // BLK
    mesh = plsc.VectorSubcoreMesh(core_axis_name="c", subcore_axis_name="s")
    cp = pltpu.CompilerParams()
    if "needs_layout_passes" in pltpu.CompilerParams.__dataclass_fields__:
        cp = dataclasses.replace(cp, needs_layout_passes=False)
    himask = jnp.int32(-65536)
    lomask = jnp.int32(0xFFFF)

    @functools.partial(
        pl.kernel,
        out_type=jax.ShapeDtypeStruct((2, NF, N_PAD), jnp.float32),
        mesh=mesh,
        compiler_params=cp,
        scratch_types=[
            pltpu.VMEM((N_PAD,), jnp.int32),
            pltpu.VMEM((N_PAD,), jnp.int32),
            pltpu.VMEM((N_PAD,), jnp.int32),
            pltpu.VMEM((N_PAD,), jnp.int32),
            pltpu.VMEM((N_PAD,), jnp.float32),
            pltpu.VMEM((N_PAD,), jnp.float32),
            pltpu.VMEM((N_PAD,), jnp.float32),
            pltpu.VMEM((N_PAD,), jnp.float32),
            pltpu.VMEM((N_PAD,), jnp.float32),
            pltpu.VMEM((N_PAD,), jnp.float32),
            pltpu.VMEM((N_PAD,), jnp.float32),
            pltpu.VMEM((N_PAD,), jnp.float32),
            pltpu.VMEM((BLK,), jnp.int32),
            pltpu.VMEM((BLK,), jnp.int32),
            pltpu.VMEM((BLK,), jnp.float32),
            pltpu.VMEM((BLK,), jnp.float32),
            pltpu.SemaphoreType.DMA,
            pltpu.SemaphoreType.DMA,
            pltpu.SemaphoreType.DMA,
            pltpu.SemaphoreType.DMA,
        ],
    )
    def k(tab_hbm, ids_hbm, ex_hbm, zeros_hbm, acc_hbm,
          t0, t1, t2, t3, al0, ah0, al1, ah1, al2, ah2, al3, ah3,
          s0, s1, e0, e1, sem0, sem1, sem2, sem3):
        tbl = [t0, t1, t2, t3]
        alo = [al0, al1, al2, al3]
        ahi = [ah0, ah1, ah2, ah3]
        cid = lax.axis_index("c")
        sid = lax.axis_index("s")
        ebase = cid * half
        base = sid * 4
        for j in range(4):
            pltpu.async_copy(tab_hbm.at[base + j], tbl[j], sem0)
            pltpu.async_copy(zeros_hbm, alo[j], sem1)
            pltpu.async_copy(zeros_hbm, ahi[j], sem1)
        for j in range(4):
            pltpu.make_async_copy(tab_hbm.at[base + j], tbl[j], sem0).wait()
            pltpu.make_async_copy(zeros_hbm, alo[j], sem1).wait()
            pltpu.make_async_copy(zeros_hbm, ahi[j], sem1).wait()

        def start(bs, be, blk, sa, sb):
            off = ebase + blk * BLK
            pltpu.async_copy(ids_hbm.at[pl.ds(off, BLK)], bs, sa)
            pltpu.async_copy(ex_hbm.at[pl.ds(off, BLK)], be, sb)

        def wait(bs, be, sa, sb):
            pltpu.make_async_copy(ids_hbm.at[pl.ds(0, BLK)], bs, sa).wait()
            pltpu.make_async_copy(ex_hbm.at[pl.ds(0, BLK)], be, sb).wait()

        def process(bs, be):
            @plsc.parallel_loop(0, BLK // 16 // GRP)
            def _(cc):
                c0 = cc * GRP
                ids_ = [bs[pl.ds((c0 + g) * 16, 16)] for g in range(GRP)]
                sis = [i & lomask for i in ids_]
                dis = [lax.shift_right_logical(i, 16) for i in ids_]
                es = [be[pl.ds((c0 + g) * 16, 16)] for g in range(GRP)]
                for j in range(4):
                    vs = [plsc.load_gather(tbl[j], [sis[g]])
                          for g in range(GRP)]
                    los = [plsc.bitcast(v << 16, jnp.float32) for v in vs]
                    his = [plsc.bitcast(v & himask, jnp.float32) for v in vs]
                    mlo = [los[g] * es[g] for g in range(GRP)]
                    mhi = [his[g] * es[g] for g in range(GRP)]
                    for g in range(GRP):
                        plsc.addupdate_scatter(alo[j], [dis[g]], mlo[g])
                    for g in range(GRP):
                        plsc.addupdate_scatter(ahi[j], [dis[g]], mhi[g])

        start(s0, e0, 0, sem0, sem1)
        start(s1, e1, 1, sem2, sem3)

        @pl.loop(0, nblk // 2)
        def _(p):
            g = p * 2
            wait(s0, e0, sem0, sem1)
            process(s0, e0)
            start(s0, e0, lax.rem(g + 2, nblk), sem0, sem1)
            wait(s1, e1, sem2, sem3)
            process(s1, e1)
            start(s1, e1, lax.rem(g + 3, nblk), sem2, sem3)

        wait(s0, e0, sem0, sem1)
        if nblk % 2 == 1:
            process(s0, e0)
        wait(s1, e1, sem2, sem3)

        for j in range(4):
            pltpu.sync_copy(alo[j], acc_hbm.at[cid, base + j])
            pltpu.sync_copy(ahi[j], acc_hbm.at[cid, base + j + NF // 2])

    return k(tabp, ids, ex, jnp.zeros((N_PAD,), jnp.float32))


def kernel(x, edge_index, W_src1, W_dst1, att_src1, att_dst1, bias1, W_l1,
           b_l1, W_src2, W_dst2, att_src2, att_dst2, bias2, W_l2, b_l2):
    n = x.shape[0]
    xT = jnp.pad(x, ((0, N_PAD - n), (0, 0))).T
    src = edge_index[0].astype(jnp.int32)
    dst = edge_index[1].astype(jnp.int32)

    tab1, att1, skt1, bmx1 = _tc_pre(
        xT, W_src1, W_dst1, att_src1.reshape(1, NF), att_dst1.reshape(1, NF),
        W_l1, b_l1.reshape(NF, 1))
    ex1, ids1, denp1 = _edge_scalar_pass(att1, src, dst, bmx1[2, :16])
    acc1 = _edge_pass(tab1, ids1, ex1)
    tab2, att2, skt2, bmx2 = _tc_mid(
        acc1, denp1, bias1.reshape(NF, 1), skt1, W_src2, W_dst2,
        att_src2.reshape(1, NF), att_dst2.reshape(1, NF), W_l2,
        b_l2.reshape(NF, 1))
    ex2, ids2, denp2 = _edge_scalar_pass(att2, src, dst, bmx2[2, :16])
    acc2 = _edge_pass(tab2, ids2, ex2)
    out = _tc_post(acc2, denp2, bias2.reshape(NF, 1), skt2)
    return out[:n]


# speedup vs baseline: 1.0588x; 1.0588x over previous
import dataclasses
import functools

import jax
import jax.numpy as jnp
from jax import lax
from jax.experimental import pallas as pl
from jax.experimental.pallas import tpu as pltpu
from jax.experimental.pallas import tpu_sc as plsc

N_PAD = 10240
R = 1024
NF = 128
TROWS = 136
BLK = 800
GRP = 5
FPW = 4


def _leaky_bound(m):
    return jnp.maximum(m, 0.2 * m)


def _pack_rows(xsT):
    lo = lax.bitcast_convert_type(
        xsT[: NF // 2, :].astype(jnp.bfloat16), jnp.uint16).astype(jnp.uint32)
    hi = lax.bitcast_convert_type(
        xsT[NF // 2:, :].astype(jnp.bfloat16), jnp.uint16).astype(jnp.uint32)
    return lax.bitcast_convert_type((hi << 16) | lo, jnp.int32)


def _pre_body(xT, Wsrc, Wdst, asrc, adst, Wl, bl, tab, att, skt, bmx):
    i = pl.program_id(0)
    xTb = xT[...]
    xsT = lax.dot_general(Wsrc[...], xTb, (((0,), (0,)), ((), ())),
                          preferred_element_type=jnp.float32)
    a_s = jnp.dot(asrc[...], xsT, preferred_element_type=jnp.float32)
    u = lax.dot_general(adst[...], Wdst[...], (((1,), (1,)), ((), ())),
                        preferred_element_type=jnp.float32)
    a_d = jnp.dot(u, xTb, preferred_element_type=jnp.float32)
    tab[...] = _pack_rows(xsT)
    att[...] = jnp.concatenate(
        [a_s, a_d, jnp.zeros((6, R), jnp.float32)], axis=0)
    skt[...] = lax.dot_general(Wl[...], xTb, (((0,), (0,)), ((), ())),
                               preferred_element_type=jnp.float32) + bl[...]

    @pl.when(i == 0)
    def _():
        bmx[...] = jnp.full((8, 128), -jnp.inf, jnp.float32)

    ms = jnp.max(a_s)
    md = jnp.max(a_d)
    bmx[0:1, :] = jnp.maximum(bmx[0:1, :], ms)
    bmx[1:2, :] = jnp.maximum(bmx[1:2, :], md)

    @pl.when(i == pl.num_programs(0) - 1)
    def _():
        m = bmx[0:1, :] + bmx[1:2, :]
        bmx[2:3, :] = _leaky_bound(m)


def _tc_pre(xT, Wsrc, Wdst, asrc2d, adst2d, Wl, blcol):
    grid = (N_PAD // R,)
    return pl.pallas_call(
        _pre_body,
        grid=grid,
        in_specs=[
            pl.BlockSpec((NF, R), lambda i: (0, i)),
            pl.BlockSpec((NF, NF), lambda i: (0, 0)),
            pl.BlockSpec((NF, NF), lambda i: (0, 0)),
            pl.BlockSpec((1, NF), lambda i: (0, 0)),
            pl.BlockSpec((1, NF), lambda i: (0, 0)),
            pl.BlockSpec((NF, NF), lambda i: (0, 0)),
            pl.BlockSpec((NF, 1), lambda i: (0, 0)),
        ],
        out_specs=[
            pl.BlockSpec((NF // 2, R), lambda i: (0, i)),
            pl.BlockSpec((8, R), lambda i: (0, i)),
            pl.BlockSpec((NF, R), lambda i: (0, i)),
            pl.BlockSpec((8, 128), lambda i: (0, 0)),
        ],
        out_shape=[
            jax.ShapeDtypeStruct((NF // 2, N_PAD), jnp.int32),
            jax.ShapeDtypeStruct((8, N_PAD), jnp.float32),
            jax.ShapeDtypeStruct((NF, N_PAD), jnp.float32),
            jax.ShapeDtypeStruct((8, 128), jnp.float32),
        ],
    )(xT, Wsrc, Wdst, asrc2d, adst2d, Wl, blcol)


def _mid_body(acc, den, b1, skt1, Wsrc, Wdst, asrc, adst, Wl, bl,
              tab, att, skt, bmx):
    i = pl.program_id(0)
    dinv = 1.0 / (jnp.sum(den[...], axis=0, keepdims=True) + 1e-16)
    hT = jnp.maximum((acc[0] + acc[1]) * dinv + b1[...] + skt1[...], 0.0)
    xsT = lax.dot_general(Wsrc[...], hT, (((0,), (0,)), ((), ())),
                          preferred_element_type=jnp.float32)
    a_s = jnp.dot(asrc[...], xsT, preferred_element_type=jnp.float32)
    u = lax.dot_general(adst[...], Wdst[...], (((1,), (1,)), ((), ())),
                        preferred_element_type=jnp.float32)
    a_d = jnp.dot(u, hT, preferred_element_type=jnp.float32)
    tab[...] = _pack_rows(xsT)
    att[...] = jnp.concatenate(
        [a_s, a_d, jnp.zeros((6, R), jnp.float32)], axis=0)
    skt[...] = lax.dot_general(Wl[...], hT, (((0,), (0,)), ((), ())),
                               preferred_element_type=jnp.float32) + bl[...]

    @pl.when(i == 0)
    def _():
        bmx[...] = jnp.full((8, 128), -jnp.inf, jnp.float32)

    ms = jnp.max(a_s)
    md = jnp.max(a_d)
    bmx[0:1, :] = jnp.maximum(bmx[0:1, :], ms)
    bmx[1:2, :] = jnp.maximum(bmx[1:2, :], md)

    @pl.when(i == pl.num_programs(0) - 1)
    def _():
        m = bmx[0:1, :] + bmx[1:2, :]
        bmx[2:3, :] = _leaky_bound(m)


def _tc_mid(acc1, den1, b1col, skt1, Wsrc, Wdst, asrc2d, adst2d, Wl, blcol):
    grid = (N_PAD // R,)
    return pl.pallas_call(
        _mid_body,
        grid=grid,
        in_specs=[
            pl.BlockSpec((2, NF, R), lambda i: (0, 0, i)),
            pl.BlockSpec((32, R), lambda i: (0, i)),
            pl.BlockSpec((NF, 1), lambda i: (0, 0)),
            pl.BlockSpec((NF, R), lambda i: (0, i)),
            pl.BlockSpec((NF, NF), lambda i: (0, 0)),
            pl.BlockSpec((NF, NF), lambda i: (0, 0)),
            pl.BlockSpec((1, NF), lambda i: (0, 0)),
            pl.BlockSpec((1, NF), lambda i: (0, 0)),
            pl.BlockSpec((NF, NF), lambda i: (0, 0)),
            pl.BlockSpec((NF, 1), lambda i: (0, 0)),
        ],
        out_specs=[
            pl.BlockSpec((NF // 2, R), lambda i: (0, i)),
            pl.BlockSpec((8, R), lambda i: (0, i)),
            pl.BlockSpec((NF, R), lambda i: (0, i)),
            pl.BlockSpec((8, 128), lambda i: (0, 0)),
        ],
        out_shape=[
            jax.ShapeDtypeStruct((NF // 2, N_PAD), jnp.int32),
            jax.ShapeDtypeStruct((8, N_PAD), jnp.float32),
            jax.ShapeDtypeStruct((NF, N_PAD), jnp.float32),
            jax.ShapeDtypeStruct((8, 128), jnp.float32),
        ],
    )(acc1, den1, b1col, skt1, Wsrc, Wdst, asrc2d, adst2d, Wl, blcol)


def _post_body(acc, den, b2, skt2, out):
    dinv = 1.0 / (jnp.sum(den[...], axis=0, keepdims=True) + 1e-16)
    outT = (acc[0] + acc[1]) * dinv + b2[...] + skt2[...]
    out[...] = outT.T


def _tc_post(acc2, den2, b2col, skt2):
    grid = (N_PAD // R,)
    return pl.pallas_call(
        _post_body,
        grid=grid,
        in_specs=[
            pl.BlockSpec((2, NF, R), lambda i: (0, 0, i)),
            pl.BlockSpec((32, R), lambda i: (0, i)),
            pl.BlockSpec((NF, 1), lambda i: (0, 0)),
            pl.BlockSpec((NF, R), lambda i: (0, i)),
        ],
        out_specs=pl.BlockSpec((R, NF), lambda i: (i, 0)),
        out_shape=jax.ShapeDtypeStruct((N_PAD, NF), jnp.float32),
    )(acc2, den2, b2col, skt2)


BLK_A = 2000
GRP_A = 5


def _edge_scalar_pass(att, src, dst, b16):
    E = src.shape[0]
    shard = E // 32
    mesh = plsc.VectorSubcoreMesh(core_axis_name="c", subcore_axis_name="s")
    cp = pltpu.CompilerParams()
    if "needs_layout_passes" in pltpu.CompilerParams.__dataclass_fields__:
        cp = dataclasses.replace(cp, needs_layout_passes=False)

    @functools.partial(
        pl.kernel,
        out_type=(jax.ShapeDtypeStruct((E,), jnp.float32),
                  jax.ShapeDtypeStruct((E,), jnp.int32),
                  jax.ShapeDtypeStruct((32, N_PAD), jnp.float32)),
        mesh=mesh,
        compiler_params=cp,
        scratch_types=[
            pltpu.VMEM((N_PAD,), jnp.float32),
            pltpu.VMEM((N_PAD,), jnp.float32),
            pltpu.VMEM((N_PAD,), jnp.float32),
            pltpu.VMEM((shard,), jnp.float32),
            pltpu.VMEM((shard,), jnp.int32),
            pltpu.VMEM((16,), jnp.float32),
            pltpu.VMEM((BLK_A,), jnp.int32),
            pltpu.VMEM((BLK_A,), jnp.int32),
            pltpu.VMEM((BLK_A,), jnp.int32),
            pltpu.VMEM((BLK_A,), jnp.int32),
            pltpu.SemaphoreType.DMA,
            pltpu.SemaphoreType.DMA,
            pltpu.SemaphoreType.DMA,
            pltpu.SemaphoreType.DMA,
        ],
    )
    def k(att_hbm, src_hbm, dst_hbm, b_hbm, ex_hbm, ids_hbm, denp_hbm,
          asrc_v, adst_v, den_v, ex_v, ids_v, b_v, s0, s1, d0, d1,
          sem0, sem1, sem2, sem3):
        wid = lax.axis_index("s") * 2 + lax.axis_index("c")
        ebase = wid * shard
        pltpu.sync_copy(att_hbm.at[0], asrc_v)
        pltpu.sync_copy(att_hbm.at[1], adst_v)
        pltpu.sync_copy(b_hbm, b_v)

        zz = jnp.zeros((16,), jnp.float32)

        @pl.loop(0, N_PAD // 16, unroll=4)
        def _(i):
            den_v[pl.ds(i * 16, 16)] = zz

        bvec = b_v[...]
        nblk = shard // BLK_A

        def start(bs, bd, blk, sa, sb):
            off = ebase + blk * BLK_A
            pltpu.async_copy(src_hbm.at[pl.ds(off, BLK_A)], bs, sa)
            pltpu.async_copy(dst_hbm.at[pl.ds(off, BLK_A)], bd, sb)

        def wait(bs, bd, sa, sb):
            pltpu.make_async_copy(src_hbm.at[pl.ds(0, BLK_A)], bs, sa).wait()
            pltpu.make_async_copy(dst_hbm.at[pl.ds(0, BLK_A)], bd, sb).wait()

        def process(bs, bd, blk):
            xbase = blk * BLK_A

            @plsc.parallel_loop(0, BLK_A // 16 // GRP_A)
            def _(cc):
                c0 = cc * GRP_A
                sis = [bs[pl.ds((c0 + g) * 16, 16)] for g in range(GRP_A)]
                dis = [bd[pl.ds((c0 + g) * 16, 16)] for g in range(GRP_A)]
                aas = [plsc.load_gather(asrc_v, [sis[g]]) for g in range(GRP_A)]
                ads = [plsc.load_gather(adst_v, [dis[g]]) for g in range(GRP_A)]
                als = [aas[g] + ads[g] for g in range(GRP_A)]
                als = [jnp.maximum(al, al * 0.2) - bvec for al in als]
                es = [jnp.exp(al) for al in als]
                for g in range(GRP_A):
                    plsc.addupdate_scatter(den_v, [dis[g]], es[g])
                    ex_v[pl.ds(xbase + (c0 + g) * 16, 16)] = es[g]
                    ids_v[pl.ds(xbase + (c0 + g) * 16, 16)] = (
                        sis[g] | (dis[g] << 16))

        start(s0, d0, 0, sem0, sem1)
        start(s1, d1, 1, sem2, sem3)

        @pl.loop(0, nblk // 2)
        def _(p):
            g = p * 2
            wait(s0, d0, sem0, sem1)
            process(s0, d0, g)
            start(s0, d0, lax.rem(g + 2, nblk), sem0, sem1)
            wait(s1, d1, sem2, sem3)
            process(s1, d1, g + 1)
            start(s1, d1, lax.rem(g + 3, nblk), sem2, sem3)

        wait(s0, d0, sem0, sem1)
        if nblk % 2 == 1:
            process(s0, d0, nblk - 1)
        wait(s1, d1, sem2, sem3)

        pltpu.sync_copy(ex_v, ex_hbm.at[pl.ds(ebase, shard)])
        pltpu.sync_copy(ids_v, ids_hbm.at[pl.ds(ebase, shard)])
        pltpu.sync_copy(den_v, denp_hbm.at[wid])

    return k(att, src, dst, b16)


def _edge_pass(tabp, ids, ex):
    E = ids.shape[0]
    half = E // 2
    nblk = half // BLK
    mesh = plsc.VectorSubcoreMesh(core_axis_name="c", subcore_axis_name="s")
    cp = pltpu.CompilerParams()
    if "needs_layout_passes" in pltpu.CompilerParams.__dataclass_fields__:
        cp = dataclasses.replace(cp, needs_layout_passes=False)
    himask = jnp.int32(-65536)
    lomask = jnp.int32(0xFFFF)

    @functools.partial(
        pl.kernel,
        out_type=jax.ShapeDtypeStruct((2, NF, N_PAD), jnp.float32),
        mesh=mesh,
        compiler_params=cp,
        scratch_types=[
            pltpu.VMEM((N_PAD,), jnp.int32),
            pltpu.VMEM((N_PAD,), jnp.int32),
            pltpu.VMEM((N_PAD,), jnp.int32),
            pltpu.VMEM((N_PAD,), jnp.int32),
            pltpu.VMEM((N_PAD,), jnp.float32),
            pltpu.VMEM((N_PAD,), jnp.float32),
            pltpu.VMEM((N_PAD,), jnp.float32),
            pltpu.VMEM((N_PAD,), jnp.float32),
            pltpu.VMEM((N_PAD,), jnp.float32),
            pltpu.VMEM((N_PAD,), jnp.float32),
            pltpu.VMEM((N_PAD,), jnp.float32),
            pltpu.VMEM((N_PAD,), jnp.float32),
            pltpu.VMEM((BLK,), jnp.int32),
            pltpu.VMEM((BLK,), jnp.int32),
            pltpu.VMEM((BLK,), jnp.float32),
            pltpu.VMEM((BLK,), jnp.float32),
            pltpu.SemaphoreType.DMA,
            pltpu.SemaphoreType.DMA,
            pltpu.SemaphoreType.DMA,
            pltpu.SemaphoreType.DMA,
        ],
    )
    def k(tab_hbm, ids_hbm, ex_hbm, acc_hbm,
          t0, t1, t2, t3, al0, ah0, al1, ah1, al2, ah2, al3, ah3,
          s0, s1, e0, e1, sem0, sem1, sem2, sem3):
        tbl = [t0, t1, t2, t3]
        alo = [al0, al1, al2, al3]
        ahi = [ah0, ah1, ah2, ah3]
        cid = lax.axis_index("c")
        sid = lax.axis_index("s")
        ebase = cid * half
        base = sid * 4
        for j in range(4):
            pltpu.sync_copy(tab_hbm.at[base + j], tbl[j])

        zz = jnp.zeros((16,), jnp.float32)

        @pl.loop(0, N_PAD // 16, unroll=4)
        def _(i):
            sl = pl.ds(i * 16, 16)
            for j in range(4):
                alo[j][sl] = zz
                ahi[j][sl] = zz

        def start(bs, be, blk, sa, sb):
            off = ebase + blk * BLK
            pltpu.async_copy(ids_hbm.at[pl.ds(off, BLK)], bs, sa)
            pltpu.async_copy(ex_hbm.at[pl.ds(off, BLK)], be, sb)

        def wait(bs, be, sa, sb):
            pltpu.make_async_copy(ids_hbm.at[pl.ds(0, BLK)], bs, sa).wait()
            pltpu.make_async_copy(ex_hbm.at[pl.ds(0, BLK)], be, sb).wait()

        def process(bs, be):
            @plsc.parallel_loop(0, BLK // 16 // GRP)
            def _(cc):
                c0 = cc * GRP
                ids_ = [bs[pl.ds((c0 + g) * 16, 16)] for g in range(GRP)]
                sis = [i & lomask for i in ids_]
                dis = [lax.shift_right_logical(i, 16) for i in ids_]
                es = [be[pl.ds((c0 + g) * 16, 16)] for g in range(GRP)]
                for j in range(4):
                    vs = [plsc.load_gather(tbl[j], [sis[g]])
                          for g in range(GRP)]
                    los = [plsc.bitcast(v << 16, jnp.float32) for v in vs]
                    his = [plsc.bitcast(v & himask, jnp.float32) for v in vs]
                    mlo = [los[g] * es[g] for g in range(GRP)]
                    mhi = [his[g] * es[g] for g in range(GRP)]
                    for g in range(GRP):
                        plsc.addupdate_scatter(alo[j], [dis[g]], mlo[g])
                    for g in range(GRP):
                        plsc.addupdate_scatter(ahi[j], [dis[g]], mhi[g])

        start(s0, e0, 0, sem0, sem1)
        start(s1, e1, 1, sem2, sem3)

        @pl.loop(0, nblk // 2)
        def _(p):
            g = p * 2
            wait(s0, e0, sem0, sem1)
            process(s0, e0)
            start(s0, e0, lax.rem(g + 2, nblk), sem0, sem1)
            wait(s1, e1, sem2, sem3)
            process(s1, e1)
            start(s1, e1, lax.rem(g + 3, nblk), sem2, sem3)

        wait(s0, e0, sem0, sem1)
        if nblk % 2 == 1:
            process(s0, e0)
        wait(s1, e1, sem2, sem3)

        for j in range(4):
            pltpu.sync_copy(alo[j], acc_hbm.at[cid, base + j])
            pltpu.sync_copy(ahi[j], acc_hbm.at[cid, base + j + NF // 2])

    return k(tabp, ids, ex)


def kernel(x, edge_index, W_src1, W_dst1, att_src1, att_dst1, bias1, W_l1,
           b_l1, W_src2, W_dst2, att_src2, att_dst2, bias2, W_l2, b_l2):
    n = x.shape[0]
    xT = jnp.pad(x, ((0, N_PAD - n), (0, 0))).T
    src = edge_index[0].astype(jnp.int32)
    dst = edge_index[1].astype(jnp.int32)

    tab1, att1, skt1, bmx1 = _tc_pre(
        xT, W_src1, W_dst1, att_src1.reshape(1, NF), att_dst1.reshape(1, NF),
        W_l1, b_l1.reshape(NF, 1))
    ex1, ids1, denp1 = _edge_scalar_pass(att1, src, dst, bmx1[2, :16])
    acc1 = _edge_pass(tab1, ids1, ex1)
    tab2, att2, skt2, bmx2 = _tc_mid(
        acc1, denp1, bias1.reshape(NF, 1), skt1, W_src2, W_dst2,
        att_src2.reshape(1, NF), att_dst2.reshape(1, NF), W_l2,
        b_l2.reshape(NF, 1))
    ex2, ids2, denp2 = _edge_scalar_pass(att2, src, dst, bmx2[2, :16])
    acc2 = _edge_pass(tab2, ids2, ex2)
    out = _tc_post(acc2, denp2, bias2.reshape(NF, 1), skt2)
    return out[:n]
